# baseline (speedup 1.0000x reference)
import jax
import jax.numpy as jnp
from jax.experimental import pallas as pl
from jax.experimental.pallas import tpu as pltpu

_B, _C, _L = 16, 128, 16384
_K, _S = 8, 4
_ITERS = 20
_W = (_L - _K) // _S + 1
_WSUB = 32
_GRP = 8
_WG = _WSUB * _GRP
_NG = 16
_NTILES = _L // _C


def _pool_kernel(q_ref, a_ref, x_ref, o_ref, s_ref):
    g = pl.program_id(1)
    half_alpha = 0.5 * jnp.exp(a_ref[...])
    t8 = _K * (2.0 * jax.nn.sigmoid(q_ref[...]) - 1.0)
    two_inv_alpha = 2.0 * jnp.exp(-a_ref[...])

    @pl.when(g == 0)
    def _():
        for t in range(_NTILES):
            s_ref[pl.ds(t * _C, _C), :] = jnp.transpose(
                x_ref[0, :, pl.ds(t * _C, _C)])
        s_ref[pl.ds(_L, 8), :] = jnp.zeros((8, _C), jnp.float32)

    def one_chunk(j):
        base = g * (_S * _WG) + j * (_S * _WSUB)
        vs = [half_alpha * s_ref[pl.ds(base + k, _WSUB, _S), :]
              for k in range(_K)]
        mn = vs[0]
        mx = vs[0]
        for t in vs[1:]:
            mn = jnp.minimum(mn, t)
            mx = jnp.maximum(mx, t)
        y = 0.5 * (mn + mx)
        d = 0.25 * (mx - mn) + 0.5
        for _i in range(_ITERS):
            ts = [jnp.tanh(y - v) for v in vs]
            while len(ts) > 1:
                ts = [ts[i] + ts[i + 1] for i in range(0, len(ts), 2)]
            th = ts[0] > t8
            y = y + jnp.where(th, -d, d)
            d = 0.5 * d
        return y * two_inv_alpha

    res = [one_chunk(j) for j in range(_GRP)]
    for h in range(2):
        blk = jnp.concatenate(res[4 * h:4 * h + 4], axis=0)
        o_ref[0, :, pl.ds(h * _C, _C)] = jnp.transpose(blk)


@jax.jit
def kernel(x, q_raw, alpha_raw):
    q2 = q_raw.reshape(1, _C)
    a2 = alpha_raw.reshape(1, _C)
    out = pl.pallas_call(
        _pool_kernel,
        grid=(_B, _NG),
        in_specs=[
            pl.BlockSpec((1, _C), lambda b, g: (0, 0)),
            pl.BlockSpec((1, _C), lambda b, g: (0, 0)),
            pl.BlockSpec((1, _C, _L), lambda b, g: (b, 0, 0)),
        ],
        out_specs=pl.BlockSpec((1, _C, _WG), lambda b, g: (b, 0, g)),
        out_shape=jax.ShapeDtypeStruct((_B, _C, _W), jnp.float32),
        scratch_shapes=[pltpu.VMEM((_L + 8, _C), jnp.float32)],
        compiler_params=pltpu.CompilerParams(
            dimension_semantics=("arbitrary", "arbitrary"),
            vmem_limit_bytes=48 * 1024 * 1024,
        ),
    )(q2, a2, x)
    return out


# speedup vs baseline: 1.0283x; 1.0283x over previous
import jax
import jax.numpy as jnp
from jax.experimental import pallas as pl
from jax.experimental.pallas import tpu as pltpu

_B, _C, _L = 16, 128, 16384
_K, _S = 8, 4
_ITERS = 20
_W = (_L - _K) // _S + 1
_WSUB = 32
_GRP = 8
_WG = _WSUB * _GRP
_NG = 16
_NTILES = _L // _C


def _pool_kernel(q_ref, a_ref, x_ref, o_ref, s_ref):
    g = pl.program_id(1)
    half_alpha = 0.5 * jnp.exp(a_ref[...])
    t8 = _K * (2.0 * jax.nn.sigmoid(q_ref[...]) - 1.0)
    two_inv_alpha = 2.0 * jnp.exp(-a_ref[...])

    @pl.when(g == 0)
    def _():
        for t in range(_NTILES):
            s_ref[pl.ds(t * _C, _C), :] = jnp.transpose(
                x_ref[0, :, pl.ds(t * _C, _C)])
        s_ref[pl.ds(_L, 8), :] = jnp.zeros((8, _C), jnp.float32)

    def one_chunk(j):
        base = g * (_S * _WG) + j * (_S * _WSUB)
        vs = [half_alpha * s_ref[pl.ds(base + k, _WSUB, _S), :]
              for k in range(_K)]
        mn = vs[0]
        mx = vs[0]
        for t in vs[1:]:
            mn = jnp.minimum(mn, t)
            mx = jnp.maximum(mx, t)
        y = 0.5 * (mn + mx)
        d = 0.25 * (mx - mn) + 0.5
        for _i in range(_ITERS):
            acc = jnp.tanh(y - vs[0])
            for v in vs[1:]:
                acc = acc + jnp.tanh(y - v)
            th = acc > t8
            y = y + jnp.where(th, -d, d)
            d = 0.5 * d
        return y * two_inv_alpha

    res = [one_chunk(j) for j in range(_GRP)]
    for h in range(2):
        blk = jnp.concatenate(res[4 * h:4 * h + 4], axis=0)
        o_ref[0, :, pl.ds(h * _C, _C)] = jnp.transpose(blk)


@jax.jit
def kernel(x, q_raw, alpha_raw):
    q2 = q_raw.reshape(1, _C)
    a2 = alpha_raw.reshape(1, _C)
    out = pl.pallas_call(
        _pool_kernel,
        grid=(_B, _NG),
        in_specs=[
            pl.BlockSpec((1, _C), lambda b, g: (0, 0)),
            pl.BlockSpec((1, _C), lambda b, g: (0, 0)),
            pl.BlockSpec((1, _C, _L), lambda b, g: (b, 0, 0)),
        ],
        out_specs=pl.BlockSpec((1, _C, _WG), lambda b, g: (b, 0, g)),
        out_shape=jax.ShapeDtypeStruct((_B, _C, _W), jnp.float32),
        scratch_shapes=[pltpu.VMEM((_L + 8, _C), jnp.float32)],
        compiler_params=pltpu.CompilerParams(
            dimension_semantics=("arbitrary", "arbitrary"),
            vmem_limit_bytes=48 * 1024 * 1024,
        ),
    )(q2, a2, x)
    return out
